# baseline (speedup 1.0000x reference)
import jax
import jax.numpy as jnp
from jax.experimental import pallas as pl
from jax.experimental.pallas import tpu as pltpu

N_TOK = 32768
IN_DIM = 128
FEAT_DIM = 128
ATTN_DIM = 64
NUM_CLASSES = 2
N_BAGS = 16

BLK = 2048
NB = N_TOK // BLK


def _mil_kernel(x_ref, sizes_ref, W1_ref, b1_ref, Wa1_ref, ba1_ref,
                Wa2_ref, ba2_ref, Wc_ref, bc_ref, out_ref, hw_ref, w_ref):
    i = pl.program_id(0)
    h = jnp.maximum(
        jnp.dot(x_ref[...], W1_ref[...], preferred_element_type=jnp.float32)
        + b1_ref[...], 0.0)
    t = jnp.tanh(
        jnp.dot(h.astype(jnp.bfloat16), Wa1_ref[...],
                preferred_element_type=jnp.float32)
        + ba1_ref[...])
    s = (jnp.dot(t, Wa2_ref[...], preferred_element_type=jnp.float32)
         + ba2_ref[...])
    c = jnp.sum(jnp.abs(Wa2_ref[...])) + jnp.abs(ba2_ref[0, 0])
    w = jnp.exp(s - c)
    w_ref[pl.ds(i * BLK, BLK), :] = w
    hw_ref[pl.ds(i * BLK, BLK), :] = h * w

    @pl.when(i == NB - 1)
    def _finalize():
        sizes = sizes_ref[...]
        tri_r = jax.lax.broadcasted_iota(jnp.int32, (N_BAGS, N_BAGS), 0)
        tri_c = jax.lax.broadcasted_iota(jnp.int32, (N_BAGS, N_BAGS), 1)
        lower = (tri_r >= tri_c).astype(jnp.float32)
        ends_f = jnp.dot(lower, sizes,
                         preferred_element_type=jnp.float32)
        ends = ends_f.astype(jnp.int32)
        starts = (ends_f - sizes).astype(jnp.int32)
        lane = jax.lax.broadcasted_iota(jnp.int32, (N_BAGS, N_TOK), 1)
        member = ((lane >= starts) & (lane < ends)).astype(jnp.float32)
        num = jnp.dot(member, hw_ref[...],
                      preferred_element_type=jnp.float32)
        den = jnp.dot(member, w_ref[...],
                      preferred_element_type=jnp.float32)
        emb = num / den
        out_ref[...] = (
            jnp.dot(emb, Wc_ref[...], preferred_element_type=jnp.float32)
            + bc_ref[...])


@jax.jit
def kernel(features, bag_sizes, W1, b1, Wa1, ba1, Wa2, ba2, Wc, bc):
    sizes_col = bag_sizes.astype(jnp.float32).reshape(N_BAGS, 1)
    features = features.astype(jnp.bfloat16)
    W1 = W1.astype(jnp.bfloat16)
    Wa1 = Wa1.astype(jnp.bfloat16)
    return pl.pallas_call(
        _mil_kernel,
        grid=(NB,),
        in_specs=[
            pl.BlockSpec((BLK, IN_DIM), lambda i: (i, 0)),
            pl.BlockSpec((N_BAGS, 1), lambda i: (0, 0)),
            pl.BlockSpec((IN_DIM, FEAT_DIM), lambda i: (0, 0)),
            pl.BlockSpec((1, FEAT_DIM), lambda i: (0, 0)),
            pl.BlockSpec((FEAT_DIM, ATTN_DIM), lambda i: (0, 0)),
            pl.BlockSpec((1, ATTN_DIM), lambda i: (0, 0)),
            pl.BlockSpec((ATTN_DIM, 1), lambda i: (0, 0)),
            pl.BlockSpec((1, 1), lambda i: (0, 0)),
            pl.BlockSpec((FEAT_DIM, NUM_CLASSES), lambda i: (0, 0)),
            pl.BlockSpec((1, NUM_CLASSES), lambda i: (0, 0)),
        ],
        out_specs=pl.BlockSpec((N_BAGS, NUM_CLASSES), lambda i: (0, 0)),
        scratch_shapes=[
            pltpu.VMEM((N_TOK, FEAT_DIM), jnp.float32),
            pltpu.VMEM((N_TOK, 1), jnp.float32),
        ],
        out_shape=jax.ShapeDtypeStruct((N_BAGS, NUM_CLASSES), jnp.float32),
    )(features, sizes_col, W1, b1.reshape(1, -1), Wa1, ba1.reshape(1, -1),
      Wa2, ba2.reshape(1, -1), Wc, bc.reshape(1, -1))


# speedup vs baseline: 1.3468x; 1.3468x over previous
import jax
import jax.numpy as jnp
from jax.experimental import pallas as pl
from jax.experimental.pallas import tpu as pltpu

N_TOK = 32768
IN_DIM = 128
FEAT_DIM = 128
ATTN_DIM = 64
NUM_CLASSES = 2
N_BAGS = 16

BLK = 2048
NB = N_TOK // BLK


def _mil_kernel(x_ref, sizes_ref, W1_ref, b1_ref, Wa1_ref, ba1_ref,
                Wa2_ref, ba2_ref, Wc_ref, bc_ref, out_ref, hw_ref, w_ref):
    i = pl.program_id(0)
    h = jnp.maximum(
        jnp.dot(x_ref[...].astype(jnp.bfloat16),
                W1_ref[...].astype(jnp.bfloat16),
                preferred_element_type=jnp.float32)
        + b1_ref[...], 0.0)
    t = jnp.tanh(
        jnp.dot(h.astype(jnp.bfloat16), Wa1_ref[...].astype(jnp.bfloat16),
                preferred_element_type=jnp.float32)
        + ba1_ref[...])
    s = (jnp.dot(t, Wa2_ref[...], preferred_element_type=jnp.float32)
         + ba2_ref[...])
    c = jnp.sum(jnp.abs(Wa2_ref[...])) + jnp.abs(ba2_ref[0, 0])
    w = jnp.exp(s - c)
    w_ref[pl.ds(i * BLK, BLK), :] = w
    hw_ref[pl.ds(i * BLK, BLK), :] = h * w

    @pl.when(i == NB - 1)
    def _finalize():
        sizes = sizes_ref[...]
        tri_r = jax.lax.broadcasted_iota(jnp.int32, (N_BAGS, N_BAGS), 0)
        tri_c = jax.lax.broadcasted_iota(jnp.int32, (N_BAGS, N_BAGS), 1)
        lower = (tri_r >= tri_c).astype(jnp.float32)
        ends_f = jnp.dot(lower, sizes,
                         preferred_element_type=jnp.float32)
        ends = ends_f.astype(jnp.int32)
        starts = (ends_f - sizes).astype(jnp.int32)
        lane = jax.lax.broadcasted_iota(jnp.int32, (N_BAGS, N_TOK), 1)
        member = ((lane >= starts) & (lane < ends)).astype(jnp.float32)
        num = jnp.dot(member, hw_ref[...],
                      preferred_element_type=jnp.float32)
        den = jnp.dot(member, w_ref[...],
                      preferred_element_type=jnp.float32)
        emb = num / den
        out_ref[...] = (
            jnp.dot(emb, Wc_ref[...], preferred_element_type=jnp.float32)
            + bc_ref[...])


@jax.jit
def kernel(features, bag_sizes, W1, b1, Wa1, ba1, Wa2, ba2, Wc, bc):
    sizes_col = bag_sizes.astype(jnp.float32).reshape(N_BAGS, 1)
    return pl.pallas_call(
        _mil_kernel,
        grid=(NB,),
        in_specs=[
            pl.BlockSpec((BLK, IN_DIM), lambda i: (i, 0)),
            pl.BlockSpec((N_BAGS, 1), lambda i: (0, 0)),
            pl.BlockSpec((IN_DIM, FEAT_DIM), lambda i: (0, 0)),
            pl.BlockSpec((1, FEAT_DIM), lambda i: (0, 0)),
            pl.BlockSpec((FEAT_DIM, ATTN_DIM), lambda i: (0, 0)),
            pl.BlockSpec((1, ATTN_DIM), lambda i: (0, 0)),
            pl.BlockSpec((ATTN_DIM, 1), lambda i: (0, 0)),
            pl.BlockSpec((1, 1), lambda i: (0, 0)),
            pl.BlockSpec((FEAT_DIM, NUM_CLASSES), lambda i: (0, 0)),
            pl.BlockSpec((1, NUM_CLASSES), lambda i: (0, 0)),
        ],
        out_specs=pl.BlockSpec((N_BAGS, NUM_CLASSES), lambda i: (0, 0)),
        scratch_shapes=[
            pltpu.VMEM((N_TOK, FEAT_DIM), jnp.float32),
            pltpu.VMEM((N_TOK, 1), jnp.float32),
        ],
        out_shape=jax.ShapeDtypeStruct((N_BAGS, NUM_CLASSES), jnp.float32),
    )(features, sizes_col, W1, b1.reshape(1, -1), Wa1, ba1.reshape(1, -1),
      Wa2, ba2.reshape(1, -1), Wc, bc.reshape(1, -1))


# speedup vs baseline: 1.5528x; 1.1529x over previous
import jax
import jax.numpy as jnp
from jax.experimental import pallas as pl
from jax.experimental.pallas import tpu as pltpu

N_TOK = 32768
IN_DIM = 128
FEAT_DIM = 128
ATTN_DIM = 64
NUM_CLASSES = 2
N_BAGS = 16

BLK = 4096
NB = N_TOK // BLK


def _mil_kernel(x_ref, sizes_ref, W1_ref, b1_ref, Wa1_ref, ba1_ref,
                Wa2_ref, ba2_ref, Wc_ref, bc_ref, out_ref,
                num_ref, den_ref):
    i = pl.program_id(0)
    h = jnp.maximum(
        jnp.dot(x_ref[...], W1_ref[...], preferred_element_type=jnp.float32)
        + b1_ref[...], 0.0)
    t = jnp.tanh(
        jnp.dot(h, Wa1_ref[...], preferred_element_type=jnp.float32)
        + ba1_ref[...])
    s = (jnp.dot(t, Wa2_ref[...], preferred_element_type=jnp.float32)
         + ba2_ref[...])
    c = jnp.sum(jnp.abs(Wa2_ref[...])) + jnp.abs(ba2_ref[0, 0])
    w = jnp.exp(s - c)

    sizes = sizes_ref[...]
    tri_r = jax.lax.broadcasted_iota(jnp.int32, (N_BAGS, N_BAGS), 0)
    tri_c = jax.lax.broadcasted_iota(jnp.int32, (N_BAGS, N_BAGS), 1)
    lower = (tri_r >= tri_c).astype(jnp.float32)
    ends_f = jnp.dot(lower, sizes,
                     preferred_element_type=jnp.float32)
    ends = ends_f.astype(jnp.int32)
    starts = (ends_f - sizes).astype(jnp.int32)
    lane = (jax.lax.broadcasted_iota(jnp.int32, (N_BAGS, BLK), 1)
            + i * BLK)
    member = ((lane >= starts) & (lane < ends)).astype(jnp.float32)
    num_p = jnp.dot(member, h * w,
                    preferred_element_type=jnp.float32)
    den_p = jnp.dot(member, w,
                    preferred_element_type=jnp.float32)

    first = (i == 0)
    num_ref[...] = jnp.where(first, num_p, num_ref[...] + num_p)
    den_ref[...] = jnp.where(first, den_p, den_ref[...] + den_p)

    @pl.when(i == NB - 1)
    def _finalize():
        emb = num_ref[...] / den_ref[...]
        out_ref[...] = (
            jnp.dot(emb, Wc_ref[...], preferred_element_type=jnp.float32)
            + bc_ref[...])


@jax.jit
def kernel(features, bag_sizes, W1, b1, Wa1, ba1, Wa2, ba2, Wc, bc):
    sizes_col = bag_sizes.astype(jnp.float32).reshape(N_BAGS, 1)
    return pl.pallas_call(
        _mil_kernel,
        grid=(NB,),
        in_specs=[
            pl.BlockSpec((BLK, IN_DIM), lambda i: (i, 0)),
            pl.BlockSpec((N_BAGS, 1), lambda i: (0, 0)),
            pl.BlockSpec((IN_DIM, FEAT_DIM), lambda i: (0, 0)),
            pl.BlockSpec((1, FEAT_DIM), lambda i: (0, 0)),
            pl.BlockSpec((FEAT_DIM, ATTN_DIM), lambda i: (0, 0)),
            pl.BlockSpec((1, ATTN_DIM), lambda i: (0, 0)),
            pl.BlockSpec((ATTN_DIM, 1), lambda i: (0, 0)),
            pl.BlockSpec((1, 1), lambda i: (0, 0)),
            pl.BlockSpec((FEAT_DIM, NUM_CLASSES), lambda i: (0, 0)),
            pl.BlockSpec((1, NUM_CLASSES), lambda i: (0, 0)),
        ],
        out_specs=pl.BlockSpec((N_BAGS, NUM_CLASSES), lambda i: (0, 0)),
        scratch_shapes=[
            pltpu.VMEM((N_BAGS, FEAT_DIM), jnp.float32),
            pltpu.VMEM((N_BAGS, 1), jnp.float32),
        ],
        out_shape=jax.ShapeDtypeStruct((N_BAGS, NUM_CLASSES), jnp.float32),
    )(features, sizes_col, W1, b1.reshape(1, -1), Wa1, ba1.reshape(1, -1),
      Wa2, ba2.reshape(1, -1), Wc, bc.reshape(1, -1))
